# baseline (speedup 1.0000x reference)
import functools

import jax
import jax.numpy as jnp
from jax import lax
from jax.experimental import pallas as pl
from jax.experimental.pallas import tpu as pltpu
from jax.experimental.pallas import tpu_sc as plsc

B = 8
N = 2048
L = 16
NCHUNK = N // L
NC = 2
NS = 16


def _fps_gather_sc(pts_planar):
    mesh = plsc.VectorSubcoreMesh(
        core_axis_name="c", subcore_axis_name="s", num_cores=NC, num_subcores=NS
    )

    @functools.partial(
        pl.kernel,
        out_type=jax.ShapeDtypeStruct((B * 3 * N,), jnp.float32),
        mesh=mesh,
        compiler_params=pltpu.CompilerParams(needs_layout_passes=False),
        scratch_types=[
            pltpu.VMEM((N,), jnp.float32),
            pltpu.VMEM((N,), jnp.float32),
            pltpu.VMEM((N,), jnp.float32),
            pltpu.VMEM((N,), jnp.float32),
            pltpu.VMEM((N,), jnp.int32),
            pltpu.VMEM((3 * N,), jnp.float32),
        ],
    )
    def k(pts_hbm, out_hbm, xr, yr, zr, dr, ir, gr):
        cid = lax.axis_index("c")
        sid = lax.axis_index("s")
        b = cid * 4 + sid

        @pl.when(sid < 4)
        def _():
            base = b * 3 * N
            pltpu.sync_copy(pts_hbm.at[pl.ds(base, N)], xr)
            pltpu.sync_copy(pts_hbm.at[pl.ds(base + N, N)], yr)
            pltpu.sync_copy(pts_hbm.at[pl.ds(base + 2 * N, N)], zr)

            lanes = lax.iota(jnp.int32, L)
            big = jnp.full((L,), 1e10, dtype=jnp.float32)

            @plsc.parallel_loop(0, N, step=L, unroll=8)
            def _init(o):
                dr[pl.ds(o, L)] = big

            ir[pl.ds(0, L)] = jnp.zeros((L,), jnp.int32)

            lane0 = (lanes == 0).astype(jnp.float32)
            px0 = jnp.full((L,), jnp.sum(xr[pl.ds(0, L)] * lane0))
            py0 = jnp.full((L,), jnp.sum(yr[pl.ds(0, L)] * lane0))
            pz0 = jnp.full((L,), jnp.sum(zr[pl.ds(0, L)] * lane0))

            def fps_body(i, carry):
                px, py, pz = carry

                rm0 = jnp.full((L,), -1.0, dtype=jnp.float32)
                ri0 = jnp.zeros((L,), jnp.int32)

                @plsc.parallel_loop(0, N, step=L, unroll=8, carry=(rm0, ri0))
                def chunk_loop(o, rc):
                    rm, ri = rc
                    dx = xr[pl.ds(o, L)] - px
                    dz = zr[pl.ds(o, L)] - pz
                    d = dx * dx + dz * dz
                    dy = yr[pl.ds(o, L)] - py
                    d = d + dy * dy
                    nd = jnp.minimum(dr[pl.ds(o, L)], d)
                    dr[pl.ds(o, L)] = nd
                    upd = nd > rm
                    rm = jnp.where(upd, nd, rm)
                    ri = jnp.where(upd, o + lanes, ri)
                    return rm, ri

                rm, ri = chunk_loop
                m = jnp.max(rm)
                cand = jnp.where(rm == m, ri, jnp.int32(1 << 30))
                nxt = jnp.min(cand)
                nxtv = jnp.full((L,), nxt, dtype=jnp.int32)
                plsc.store_scatter(
                    ir, [jnp.full((L,), i, dtype=jnp.int32)], nxtv,
                    mask=lanes == 0,
                )
                return (
                    plsc.load_gather(xr, [nxtv]),
                    plsc.load_gather(yr, [nxtv]),
                    plsc.load_gather(zr, [nxtv]),
                )

            lax.fori_loop(1, N, fps_body, (px0, py0, pz0))

            @plsc.parallel_loop(0, N, step=L, unroll=4)
            def _gather(o):
                iv = ir[pl.ds(o, L)]
                dst = (o + lanes) * 3
                plsc.store_scatter(gr, [dst], plsc.load_gather(xr, [iv]))
                plsc.store_scatter(gr, [dst + 1], plsc.load_gather(yr, [iv]))
                plsc.store_scatter(gr, [dst + 2], plsc.load_gather(zr, [iv]))
            pltpu.sync_copy(gr, out_hbm.at[pl.ds(b * 3 * N, 3 * N)])

    return k(pts_planar)


def _mlp_tc(x, W1, b1, W2, b2, W3, b3, W4, b4):
    M = B * N
    BLK = 2048

    def body(xr, w1r, b1r, w2r, b2r, w3r, b3r, w4r, b4r, outr):
        x_ = xr[...]
        h = jnp.maximum(
            jnp.dot(x_, w1r[...], preferred_element_type=jnp.float32) + b1r[...], 0.0
        )
        h = jnp.maximum(
            jnp.dot(h, w2r[...], preferred_element_type=jnp.float32) + b2r[...], 0.0
        )
        h = jnp.maximum(
            jnp.dot(h, w3r[...], preferred_element_type=jnp.float32) + b3r[...], 0.0
        )
        outr[...] = x_ + jnp.dot(h, w4r[...], preferred_element_type=jnp.float32) + b4r[...]

    full = lambda shape: pl.BlockSpec(shape, lambda i: (0,) * len(shape))
    return pl.pallas_call(
        body,
        grid=(M // BLK,),
        in_specs=[
            pl.BlockSpec((BLK, 3), lambda i: (i, 0)),
            full((3, 128)), full((1, 128)),
            full((128, 256)), full((1, 256)),
            full((256, 128)), full((1, 128)),
            full((128, 3)), full((1, 3)),
        ],
        out_specs=pl.BlockSpec((BLK, 3), lambda i: (i, 0)),
        out_shape=jax.ShapeDtypeStruct((M, 3), jnp.float32),
    )(x, W1, b1.reshape(1, -1), W2, b2.reshape(1, -1),
      W3, b3.reshape(1, -1), W4, b4.reshape(1, -1))


def kernel(pseudo_points, gt_points, W1, b1, W2, b2, W3, b3, W4, b4):
    del gt_points
    pts_planar = jnp.transpose(pseudo_points, (0, 2, 1)).reshape(-1)
    sampled = _fps_gather_sc(pts_planar).reshape(B * N, 3)
    refined = _mlp_tc(sampled, W1, b1, W2, b2, W3, b3, W4, b4)
    return refined.reshape(B, N, 3)


# speedup vs baseline: 1.1277x; 1.1277x over previous
import functools

import jax
import jax.numpy as jnp
from jax import lax
from jax.experimental import pallas as pl
from jax.experimental.pallas import tpu as pltpu
from jax.experimental.pallas import tpu_sc as plsc

B = 8
N = 2048
L = 16
NC = 2
NS = 16
NM = 4
SL = N // NM


def _fps_gather_sc(pts_planar):
    mesh = plsc.VectorSubcoreMesh(
        core_axis_name="c", subcore_axis_name="s", num_cores=NC, num_subcores=NS
    )

    @functools.partial(
        pl.kernel,
        out_type=jax.ShapeDtypeStruct((B * 3 * N,), jnp.float32),
        mesh=mesh,
        compiler_params=pltpu.CompilerParams(needs_layout_passes=False),
        scratch_types=[
            pltpu.VMEM((N,), jnp.float32),
            pltpu.VMEM((N,), jnp.float32),
            pltpu.VMEM((N,), jnp.float32),
            pltpu.VMEM((SL,), jnp.float32),
            pltpu.VMEM((N,), jnp.int32),
            pltpu.VMEM((3 * SL,), jnp.float32),
            pltpu.VMEM((2 * L,), jnp.float32),
            pltpu.VMEM((NM * 2 * L,), jnp.float32),
            pltpu.VMEM_SHARED((2 * 4 * NM * 2 * L,), jnp.float32),
        ],
    )
    def k(pts_hbm, out_hbm, xr, yr, zr, dr, ir, gr, stg, rb, shr):
        cid = lax.axis_index("c")
        sid = lax.axis_index("s")
        g = sid // NM
        m = sid % NM
        b = cid * 4 + g
        mbase = m * SL

        base = b * 3 * N
        pltpu.sync_copy(pts_hbm.at[pl.ds(base, N)], xr)
        pltpu.sync_copy(pts_hbm.at[pl.ds(base + N, N)], yr)
        pltpu.sync_copy(pts_hbm.at[pl.ds(base + 2 * N, N)], zr)

        lanes = lax.iota(jnp.int32, L)
        big = jnp.full((L,), 1e10, dtype=jnp.float32)

        @plsc.parallel_loop(0, SL, step=L, unroll=8)
        def _init(o):
            dr[pl.ds(o, L)] = big

        ir[pl.ds(0, L)] = jnp.zeros((L,), jnp.int32)

        lane0 = (lanes == 0).astype(jnp.float32)
        px0 = jnp.full((L,), jnp.sum(xr[pl.ds(0, L)] * lane0))
        py0 = jnp.full((L,), jnp.sum(yr[pl.ds(0, L)] * lane0))
        pz0 = jnp.full((L,), jnp.sum(zr[pl.ds(0, L)] * lane0))

        def fps_body(i, carry):
            px, py, pz = carry

            rm0 = jnp.full((L,), -1.0, dtype=jnp.float32)
            ri0 = jnp.zeros((L,), jnp.int32)

            @plsc.parallel_loop(0, SL, step=L, unroll=8, carry=(rm0, ri0))
            def chunk_loop(o, rc):
                rm, ri = rc
                dx = xr[pl.ds(mbase + o, L)] - px
                dz = zr[pl.ds(mbase + o, L)] - pz
                d = dx * dx + dz * dz
                dy = yr[pl.ds(mbase + o, L)] - py
                d = d + dy * dy
                nd = jnp.minimum(dr[pl.ds(o, L)], d)
                dr[pl.ds(o, L)] = nd
                upd = nd > rm
                rm = jnp.where(upd, nd, rm)
                ri = jnp.where(upd, mbase + o + lanes, ri)
                return rm, ri

            rm, ri = chunk_loop
            stg[pl.ds(0, L)] = rm
            stg[pl.ds(L, L)] = plsc.bitcast(ri, jnp.float32)
            par = lax.rem(i, 2)
            slot = ((par * 4 + g) * NM + m) * (2 * L)
            pltpu.sync_copy(stg, shr.at[pl.ds(slot, 2 * L)])
            plsc.subcore_barrier()
            gbase = (par * 4 + g) * (NM * 2 * L)
            pltpu.sync_copy(shr.at[pl.ds(gbase, NM * 2 * L)], rb)

            rms = [rb[pl.ds(kk * 2 * L, L)] for kk in range(NM)]
            ris = [plsc.bitcast(rb[pl.ds(kk * 2 * L + L, L)], jnp.int32)
                   for kk in range(NM)]
            mx = jnp.maximum(jnp.maximum(rms[0], rms[1]),
                             jnp.maximum(rms[2], rms[3]))
            ms = jnp.max(mx)
            bigi = jnp.int32(1 << 30)
            cs = [jnp.where(rms[kk] == ms, ris[kk], bigi) for kk in range(NM)]
            c = jnp.minimum(jnp.minimum(cs[0], cs[1]),
                            jnp.minimum(cs[2], cs[3]))
            nxt = jnp.min(c)
            nxtv = jnp.full((L,), nxt, dtype=jnp.int32)
            plsc.store_scatter(
                ir, [jnp.full((L,), i, dtype=jnp.int32)], nxtv,
                mask=lanes == 0,
            )
            return (
                plsc.load_gather(xr, [nxtv]),
                plsc.load_gather(yr, [nxtv]),
                plsc.load_gather(zr, [nxtv]),
            )

        lax.fori_loop(1, N, fps_body, (px0, py0, pz0))

        @plsc.parallel_loop(0, SL, step=L, unroll=4)
        def _gather(o):
            iv = ir[pl.ds(mbase + o, L)]
            dst = (o + lanes) * 3
            plsc.store_scatter(gr, [dst], plsc.load_gather(xr, [iv]))
            plsc.store_scatter(gr, [dst + 1], plsc.load_gather(yr, [iv]))
            plsc.store_scatter(gr, [dst + 2], plsc.load_gather(zr, [iv]))

        pltpu.sync_copy(gr, out_hbm.at[pl.ds(b * 3 * N + mbase * 3, 3 * SL)])

    return k(pts_planar)


def _mlp_tc(x, W1, b1, W2, b2, W3, b3, W4, b4):
    M = B * N
    BLK = 2048

    def body(xr, w1r, b1r, w2r, b2r, w3r, b3r, w4r, b4r, outr):
        x_ = xr[...]
        h = jnp.maximum(
            jnp.dot(x_, w1r[...], preferred_element_type=jnp.float32) + b1r[...], 0.0
        )
        h = jnp.maximum(
            jnp.dot(h, w2r[...], preferred_element_type=jnp.float32) + b2r[...], 0.0
        )
        h = jnp.maximum(
            jnp.dot(h, w3r[...], preferred_element_type=jnp.float32) + b3r[...], 0.0
        )
        outr[...] = x_ + jnp.dot(h, w4r[...], preferred_element_type=jnp.float32) + b4r[...]

    full = lambda shape: pl.BlockSpec(shape, lambda i: (0,) * len(shape))
    return pl.pallas_call(
        body,
        grid=(M // BLK,),
        in_specs=[
            pl.BlockSpec((BLK, 3), lambda i: (i, 0)),
            full((3, 128)), full((1, 128)),
            full((128, 256)), full((1, 256)),
            full((256, 128)), full((1, 128)),
            full((128, 3)), full((1, 3)),
        ],
        out_specs=pl.BlockSpec((BLK, 3), lambda i: (i, 0)),
        out_shape=jax.ShapeDtypeStruct((M, 3), jnp.float32),
    )(x, W1, b1.reshape(1, -1), W2, b2.reshape(1, -1),
      W3, b3.reshape(1, -1), W4, b4.reshape(1, -1))


def kernel(pseudo_points, gt_points, W1, b1, W2, b2, W3, b3, W4, b4):
    del gt_points
    pts_planar = jnp.transpose(pseudo_points, (0, 2, 1)).reshape(-1)
    sampled = _fps_gather_sc(pts_planar).reshape(B * N, 3)
    refined = _mlp_tc(sampled, W1, b1, W2, b2, W3, b3, W4, b4)
    return refined.reshape(B, N, 3)
